# Initial kernel scaffold; baseline (speedup 1.0000x reference)
import jax
import jax.numpy as jnp
from jax.experimental import pallas as pl


def kernel(emb_diag, emb_proc, emb_med, diag_med_weights, proc_med_weights, W1, root1, b1, W2, root2, b2):
    raise NotImplementedError("write your pallas kernel here")


# speedup vs baseline: 6176.4688x; 6176.4688x over previous
import jax
import jax.numpy as jnp
from jax.experimental import pallas as pl

N_DIAG = 1000
N_PROC = 1000
N_MED = 300
D = 128
LEVELS = 5
N = N_DIAG + N_PROC + N_MED
N_PAD = 2304
M_PAD = 304
MED0 = N_DIAG + N_PROC


def _rgcn2_kernel(x_ref, dw_ref, pw_ref, w1_ref, r1_ref, b1_ref,
                  w2_ref, r2_ref, b2_ref, out_ref):
    x = x_ref[...]
    dw = dw_ref[...]
    pw = pw_ref[...]

    izd = jnp.min(jnp.where(dw == 0.0, 1.0, 0.0))
    izp = jnp.min(jnp.where(pw == 0.0, 1.0, 0.0))

    def scaled_masks(w, iz):
        out = []
        for i in range(1, LEVELS + 1):
            m = jnp.where((w > i / LEVELS) & (w <= (i + 1) / LEVELS),
                          1.0 - iz, 0.0)
            c = jnp.sum(m, axis=1, keepdims=True)
            out.append(m / jnp.maximum(c, 1.0))
        return out

    ad = scaled_masks(dw, izd)
    ap = scaled_masks(pw, izp)

    rows = jax.lax.broadcasted_iota(jnp.int32, (N_PAD, 1), 0)
    sel_all = izd * jnp.ones((N_PAD, 1), jnp.float32)
    sel_tail = izp * jnp.where(rows >= N_DIAG, 1.0, 0.0)

    def layer(xc, w, root, bias):
        xm = xc[MED0:MED0 + M_PAD]
        agg_d = jnp.zeros((N_DIAG, D), jnp.float32)
        agg_p = jnp.zeros((N_PROC, D), jnp.float32)
        for i in range(LEVELS):
            hd = jnp.dot(xm, w[1 + i], preferred_element_type=jnp.float32)
            agg_d = agg_d + jnp.dot(ad[i], hd,
                                    preferred_element_type=jnp.float32)
            hp = jnp.dot(xm, w[LEVELS + 2 + i],
                         preferred_element_type=jnp.float32)
            agg_p = agg_p + jnp.dot(ap[i], hp,
                                    preferred_element_type=jnp.float32)
        agg = jnp.concatenate(
            [agg_d, agg_p, jnp.zeros((N_PAD - N_DIAG - N_PROC, D),
                                     jnp.float32)], axis=0)
        h0 = jnp.dot(xc[MED0:MED0 + 1], w[0],
                     preferred_element_type=jnp.float32)
        h6 = jnp.dot(xc[MED0:MED0 + 1], w[LEVELS + 1],
                     preferred_element_type=jnp.float32)
        agg = agg + sel_all * h0 + sel_tail * h6
        return agg + jnp.dot(xc, root,
                             preferred_element_type=jnp.float32) + bias

    out1 = jax.nn.relu(layer(x, w1_ref[...], r1_ref[...], b1_ref[...]))
    out_ref[...] = layer(out1, w2_ref[...], r2_ref[...], b2_ref[...])


def _run(x, dw, pw, w1, r1, b1, w2, r2, b2, interpret=False):
    return pl.pallas_call(
        _rgcn2_kernel,
        out_shape=jax.ShapeDtypeStruct((N_PAD, D), jnp.float32),
        interpret=interpret,
    )(x, dw, pw, w1, r1, b1, w2, r2, b2)


@jax.jit
def kernel(emb_diag, emb_proc, emb_med, diag_med_weights, proc_med_weights,
           W1, root1, b1, W2, root2, b2):
    x = jnp.concatenate([emb_diag[0], emb_proc[0], emb_med[0]], axis=0)
    x = jnp.pad(x, ((0, N_PAD - N), (0, 0)))
    dw = jnp.pad(diag_med_weights, ((0, 0), (0, M_PAD - N_MED)))
    pw = jnp.pad(proc_med_weights, ((0, 0), (0, M_PAD - N_MED)))
    out = _run(x, dw, pw, W1, root1, b1.reshape(1, D),
               W2, root2, b2.reshape(1, D))
    return (out[:N_DIAG][None],
            out[N_DIAG:N_DIAG + N_PROC][None],
            out[MED0:MED0 + N_MED][None])


# speedup vs baseline: 8442.9934x; 1.3670x over previous
import jax
import jax.numpy as jnp
from jax.experimental import pallas as pl

N_DIAG = 1000
N_PROC = 1000
N_MED = 300
D = 128
LEVELS = 5
M_PAD = 384


def _rgcn2_kernel(ed_ref, ep_ref, em_ref, dw_ref, pw_ref,
                  w1_ref, r1_ref, b1_ref, w2_ref, r2_ref, b2_ref,
                  od_ref, op_ref, om_ref):
    ed = ed_ref[0]
    ep = ep_ref[0]
    xm1 = em_ref[...]
    dw = dw_ref[...]
    pw = pw_ref[...]

    izd = jnp.min(jnp.where(dw == 0.0, 1.0, 0.0))
    izp = jnp.min(jnp.where(pw == 0.0, 1.0, 0.0))

    def a_cat(w, iz):
        parts = []
        for i in range(1, LEVELS + 1):
            m = jnp.where((w > i / LEVELS) & (w <= (i + 1) / LEVELS),
                          1.0 - iz, 0.0)
            c = jnp.sum(m, axis=1, keepdims=True)
            parts.append(m / jnp.maximum(c, 1.0))
        return jnp.concatenate(parts, axis=1)

    ad = a_cat(dw, izd)
    ap = a_cat(pw, izp)

    def layer(xd, xp, xm, w, root, bias):
        hd = jnp.concatenate(
            [jnp.dot(xm, w[1 + i], preferred_element_type=jnp.float32)
             for i in range(LEVELS)], axis=0)
        hp = jnp.concatenate(
            [jnp.dot(xm, w[LEVELS + 2 + i], preferred_element_type=jnp.float32)
             for i in range(LEVELS)], axis=0)
        agg_d = jnp.dot(ad, hd, preferred_element_type=jnp.float32)
        agg_p = jnp.dot(ap, hp, preferred_element_type=jnp.float32)
        h0 = izd * jnp.dot(xm[0:1], w[0], preferred_element_type=jnp.float32)
        h6 = izp * jnp.dot(xm[0:1], w[LEVELS + 1],
                           preferred_element_type=jnp.float32)
        out_d = agg_d + h0 + bias + jnp.dot(
            xd, root, preferred_element_type=jnp.float32)
        out_p = agg_p + h0 + h6 + bias + jnp.dot(
            xp, root, preferred_element_type=jnp.float32)
        out_m = h0 + h6 + bias + jnp.dot(
            xm, root, preferred_element_type=jnp.float32)
        return out_d, out_p, out_m

    d1, p1, m1 = layer(ed, ep, xm1, w1_ref[...], r1_ref[...], b1_ref[...])
    d1 = jax.nn.relu(d1)
    p1 = jax.nn.relu(p1)
    m1 = jax.nn.relu(m1)
    d2, p2, m2 = layer(d1, p1, m1, w2_ref[...], r2_ref[...], b2_ref[...])
    od_ref[0] = d2
    op_ref[0] = p2
    om_ref[0] = m2[:N_MED]


def _run(ed, ep, em, dw, pw, w1, r1, b1, w2, r2, b2, interpret=False):
    return pl.pallas_call(
        _rgcn2_kernel,
        out_shape=(
            jax.ShapeDtypeStruct((1, N_DIAG, D), jnp.float32),
            jax.ShapeDtypeStruct((1, N_PROC, D), jnp.float32),
            jax.ShapeDtypeStruct((1, N_MED, D), jnp.float32),
        ),
        interpret=interpret,
    )(ed, ep, em, dw, pw, w1, r1, b1, w2, r2, b2)


@jax.jit
def kernel(emb_diag, emb_proc, emb_med, diag_med_weights, proc_med_weights,
           W1, root1, b1, W2, root2, b2):
    pad_m = M_PAD - N_MED
    em = jnp.pad(emb_med[0], ((0, pad_m), (0, 0)))
    dw = jnp.pad(diag_med_weights, ((0, 0), (0, pad_m)))
    pw = jnp.pad(proc_med_weights, ((0, 0), (0, pad_m)))
    return _run(emb_diag, emb_proc, em, dw, pw,
                W1, root1, b1.reshape(1, D), W2, root2, b2.reshape(1, D))
